# Initial kernel scaffold; baseline (speedup 1.0000x reference)
import jax
import jax.numpy as jnp
from jax.experimental import pallas as pl


def kernel(x, edge_index, W1, b1, W2, b2, W3, b3, lin1_W, lin1_b, lin2_W, lin2_b):
    raise NotImplementedError("write your pallas kernel here")


# speedup vs baseline: 13.3638x; 13.3638x over previous
import functools

import jax
import jax.numpy as jnp
from jax import lax
from jax.experimental import pallas as pl
from jax.experimental.pallas import tpu as pltpu
from jax.experimental.pallas import tpu_sc as plsc

NC = 2
NS = 16
CH = 128
DEGW = 16


def _sc_degree(n_pad: int, e_pad: int, hp: int):
    ept = e_pad // (NC * NS)
    nch = ept // CH
    rpt = n_pad // NS
    mesh = plsc.VectorSubcoreMesh(core_axis_name="c", subcore_axis_name="s")

    @functools.partial(
        pl.kernel,
        out_type=jax.ShapeDtypeStruct((NC, n_pad, hp), jnp.float32),
        mesh=mesh,
        scratch_types=[
            pltpu.VMEM((CH,), jnp.int32),
            pltpu.VMEM((CH, hp), jnp.float32),
            pltpu.VMEM_SHARED((n_pad, hp), jnp.float32),
        ],
    )
    def k(dst_hbm, out_hbm, didx, rows, acc, *, nch=nch, rpt=rpt, ept=ept):
        cid = lax.axis_index("c")
        sid = lax.axis_index("s")
        zeros16 = jnp.zeros((16,), jnp.float32)
        ones16 = jnp.full((16,), 1.0, jnp.float32)

        def fill(val):
            def f(i, _):
                rows[i // (hp // 16), pl.ds((i % (hp // 16)) * 16, 16)] = val
                return 0
            lax.fori_loop(0, CH * (hp // 16), f, 0)

        fill(zeros16)

        def zero_acc(i, _):
            pltpu.sync_copy(rows, acc.at[pl.ds(sid * rpt + i * CH, CH)])
            return 0

        lax.fori_loop(0, rpt // CH, zero_acc, 0)
        fill(ones16)
        plsc.subcore_barrier()
        ebase = (cid * NS + sid) * ept

        def body(ci, _):
            pltpu.sync_copy(dst_hbm.at[pl.ds(ebase + ci * CH, CH)], didx)
            pltpu.sync_copy(rows, acc.at[didx], add=True)
            return 0

        lax.fori_loop(0, nch, body, 0)
        plsc.subcore_barrier()
        pltpu.sync_copy(
            acc.at[pl.ds(sid * rpt, rpt)],
            out_hbm.at[cid, pl.ds(sid * rpt, rpt)],
        )

    return k


def _sc_scatter(n_pad: int, e_pad: int, hp: int):
    ept = e_pad // (NC * NS)
    nch = ept // CH
    rpt = n_pad // NS
    mesh = plsc.VectorSubcoreMesh(core_axis_name="c", subcore_axis_name="s")

    @functools.partial(
        pl.kernel,
        out_type=jax.ShapeDtypeStruct((NC, n_pad, hp), jnp.float32),
        mesh=mesh,
        scratch_types=[
            pltpu.VMEM((CH,), jnp.int32),
            pltpu.VMEM((CH,), jnp.int32),
            pltpu.VMEM((CH, hp), jnp.float32),
            pltpu.VMEM_SHARED((n_pad, hp), jnp.float32),
            pltpu.SemaphoreType.DMA,
        ],
    )
    def k(y_hbm, src_hbm, dst_hbm, out_hbm, sidx, didx, rows, acc, sem,
          *, nch=nch, rpt=rpt, ept=ept):
        cid = lax.axis_index("c")
        sid = lax.axis_index("s")
        zeros16 = jnp.zeros((16,), jnp.float32)

        def fill(i, _):
            r = i // (hp // 16)
            j = i % (hp // 16)
            rows[r, pl.ds(j * 16, 16)] = zeros16
            return 0

        lax.fori_loop(0, CH * (hp // 16), fill, 0)

        def zero_acc(i, _):
            pltpu.sync_copy(rows, acc.at[pl.ds(sid * rpt + i * CH, CH)])
            return 0

        lax.fori_loop(0, rpt // CH, zero_acc, 0)
        plsc.subcore_barrier()
        ebase = (cid * NS + sid) * ept

        def body(ci, _):
            off = ebase + ci * CH
            pltpu.sync_copy(src_hbm.at[pl.ds(off, CH)], sidx)
            pltpu.async_copy(y_hbm.at[sidx], rows, sem).wait()
            pltpu.sync_copy(dst_hbm.at[pl.ds(off, CH)], didx)
            pltpu.sync_copy(rows, acc.at[didx], add=True)
            return 0

        lax.fori_loop(0, nch, body, 0)
        plsc.subcore_barrier()
        pltpu.sync_copy(
            acc.at[pl.ds(sid * rpt, rpt)],
            out_hbm.at[cid, pl.ds(sid * rpt, rpt)],
        )

    return k


def _tc_stage0(x_pad, w1p, degp):
    n_pad = x_pad.shape[0]
    hp = w1p.shape[1]

    def body(x_ref, w_ref, deg_ref, y_ref, dinv_ref):
        deg = deg_ref[0, :, 0:DEGW] + deg_ref[1, :, 0:DEGW] + 1.0
        dinv = lax.rsqrt(deg)
        dinv_ref[...] = dinv
        xw = jnp.dot(x_ref[...], w_ref[...], preferred_element_type=jnp.float32)
        y_ref[...] = dinv[:, 0:1] * xw

    return pl.pallas_call(
        body,
        out_shape=(
            jax.ShapeDtypeStruct((n_pad, hp), jnp.float32),
            jax.ShapeDtypeStruct((n_pad, DEGW), jnp.float32),
        ),
    )(x_pad, w1p, degp)


def _tc_mid(zp, y, dinv16, bp, wnext):
    n_pad, hp = y.shape

    def body(z_ref, y_ref, dinv_ref, b_ref, w_ref, out_ref):
        dinv = dinv_ref[:, 0:1]
        z = z_ref[0] + z_ref[1] + y_ref[...]
        h = jnp.maximum(dinv * z + b_ref[...], 0.0)
        out_ref[...] = dinv * jnp.dot(h, w_ref[...],
                                      preferred_element_type=jnp.float32)

    return pl.pallas_call(
        body,
        out_shape=jax.ShapeDtypeStruct((n_pad, hp), jnp.float32),
    )(zp, y, dinv16, bp, wnext)


def _tc_stage3(zp, y, dinv16, bp, l1w, l1b, l2w, l2b, n_real):
    n_pad, hp = y.shape
    cp = l2w.shape[1]

    def body(z_ref, y_ref, dinv_ref, b_ref, w1_ref, b1_ref, w2_ref, b2_ref,
             out_ref):
        dinv = dinv_ref[:, 0:1]
        z = z_ref[0] + z_ref[1] + y_ref[...]
        h = jnp.maximum(dinv * z + b_ref[...], 0.0)
        ridx = lax.broadcasted_iota(jnp.int32, h.shape, 0)
        h = jnp.where(ridx < n_real, h, 0.0)
        g = jnp.sum(h, axis=0, keepdims=True)
        g1 = jnp.maximum(
            jnp.dot(g, w1_ref[...], preferred_element_type=jnp.float32)
            + b1_ref[...], 0.0)
        out_ref[...] = (jnp.dot(g1, w2_ref[...],
                                preferred_element_type=jnp.float32)
                        + b2_ref[...])

    return pl.pallas_call(
        body,
        out_shape=jax.ShapeDtypeStruct((1, cp), jnp.float32),
    )(zp, y, dinv16, bp, l1w, l1b, l2w, l2b)


def kernel(x, edge_index, W1, b1, W2, b2, W3, b3, lin1_W, lin1_b, lin2_W,
           lin2_b):
    n, _ = x.shape
    e = edge_index.shape[1]
    h = W1.shape[1]
    h2 = lin1_W.shape[1]
    c = lin2_W.shape[1]

    n_pad = ((n + NS * CH - 1) // (NS * CH)) * (NS * CH)
    hp = ((h + 127) // 128) * 128
    e_pad = ((e + NC * NS * CH - 1) // (NC * NS * CH)) * (NC * NS * CH)
    h2p = ((h2 + 7) // 8) * 8
    cp = ((c + 7) // 8) * 8

    x_pad = jnp.pad(x, ((0, n_pad - n), (0, 0)))
    pad_cnt = e_pad - e
    pad_idx = (n + jnp.arange(pad_cnt, dtype=jnp.int32) % (n_pad - n)
               ).astype(jnp.int32)
    src = jnp.concatenate([edge_index[0], pad_idx])
    dst = jnp.concatenate([edge_index[1], pad_idx])
    w1p = jnp.pad(W1, ((0, 0), (0, hp - h)))
    w2p = jnp.pad(W2, ((0, hp - h), (0, hp - h)))
    w3p = jnp.pad(W3, ((0, hp - h), (0, hp - h)))
    b1p = jnp.pad(b1, (0, hp - h)).reshape(1, hp)
    b2p = jnp.pad(b2, (0, hp - h)).reshape(1, hp)
    b3p = jnp.pad(b3, (0, hp - h)).reshape(1, hp)
    l1wp = jnp.pad(lin1_W, ((0, hp - h), (0, h2p - h2)))
    l1bp = jnp.pad(lin1_b, (0, h2p - h2)).reshape(1, h2p)
    l2wp = jnp.pad(lin2_W, ((0, h2p - h2), (0, cp - c)))
    l2bp = jnp.pad(lin2_b, (0, cp - c)).reshape(1, cp)

    scatter = _sc_scatter(n_pad, e_pad, hp)

    degp = _sc_degree(n_pad, e_pad, hp)(dst)
    y1, dinv16 = _tc_stage0(x_pad, w1p, degp)
    z1 = scatter(y1, src, dst)
    y2 = _tc_mid(z1, y1, dinv16, b1p, w2p)
    z2 = scatter(y2, src, dst)
    y3 = _tc_mid(z2, y2, dinv16, b2p, w3p)
    z3 = scatter(y3, src, dst)
    outp = _tc_stage3(z3, y3, dinv16, b3p, l1wp, l1bp, l2wp, l2bp, n)
    return outp[:, :c]


# speedup vs baseline: 22.2617x; 1.6658x over previous
import functools

import jax
import jax.numpy as jnp
from jax import lax
from jax.experimental import pallas as pl
from jax.experimental.pallas import tpu as pltpu
from jax.experimental.pallas import tpu_sc as plsc

NC = 2
NS = 16
CH = 128
DEGW = 16


def _sc_degree(n_pad: int, e_pad: int, hp: int):
    ept = e_pad // (NC * NS)
    nch = ept // CH
    rpt = n_pad // NS
    mesh = plsc.VectorSubcoreMesh(core_axis_name="c", subcore_axis_name="s")

    @functools.partial(
        pl.kernel,
        out_type=jax.ShapeDtypeStruct((NC, n_pad, hp), jnp.float32),
        mesh=mesh,
        scratch_types=[
            pltpu.VMEM((CH,), jnp.int32),
            pltpu.VMEM((CH, hp), jnp.float32),
            pltpu.VMEM_SHARED((n_pad, hp), jnp.float32),
        ],
    )
    def k(dst_hbm, out_hbm, didx, rows, acc, *, nch=nch, rpt=rpt, ept=ept):
        cid = lax.axis_index("c")
        sid = lax.axis_index("s")
        zeros16 = jnp.zeros((16,), jnp.float32)
        ones16 = jnp.full((16,), 1.0, jnp.float32)

        def fill(val):
            def f(i, _):
                rows[i // (hp // 16), pl.ds((i % (hp // 16)) * 16, 16)] = val
                return 0
            lax.fori_loop(0, CH * (hp // 16), f, 0)

        fill(zeros16)

        def zero_acc(i, _):
            pltpu.sync_copy(rows, acc.at[pl.ds(sid * rpt + i * CH, CH)])
            return 0

        lax.fori_loop(0, rpt // CH, zero_acc, 0)
        fill(ones16)
        plsc.subcore_barrier()
        ebase = (cid * NS + sid) * ept

        def body(ci, _):
            pltpu.sync_copy(dst_hbm.at[pl.ds(ebase + ci * CH, CH)], didx)
            pltpu.sync_copy(rows, acc.at[didx], add=True)
            return 0

        lax.fori_loop(0, nch, body, 0)
        plsc.subcore_barrier()
        pltpu.sync_copy(
            acc.at[pl.ds(sid * rpt, rpt)],
            out_hbm.at[cid, pl.ds(sid * rpt, rpt)],
        )

    return k


def _sc_scatter(n_pad: int, e_pad: int, hp: int):
    ept = e_pad // (NC * NS)
    nch = ept // CH
    rpt = n_pad // NS
    mesh = plsc.VectorSubcoreMesh(core_axis_name="c", subcore_axis_name="s")

    @functools.partial(
        pl.kernel,
        out_type=jax.ShapeDtypeStruct((NC, n_pad, hp), jnp.float32),
        mesh=mesh,
        scratch_types=[
            pltpu.VMEM((2, CH), jnp.int32),
            pltpu.VMEM((2, CH), jnp.int32),
            pltpu.VMEM((CH, hp), jnp.float32),
            pltpu.VMEM((CH, hp), jnp.float32),
            pltpu.VMEM_SHARED((n_pad, hp), jnp.float32),
            pltpu.SemaphoreType.DMA,
            pltpu.SemaphoreType.DMA,
        ],
    )
    def k(y_hbm, eidx_hbm, out_hbm, idx0, idx1, rows0, rows1, acc,
          sem0, sem1, *, nch=nch, rpt=rpt):
        cid = lax.axis_index("c")
        sid = lax.axis_index("s")
        zeros16 = jnp.zeros((16,), jnp.float32)

        def fill(i, _):
            rows0[i // (hp // 16), pl.ds((i % (hp // 16)) * 16, 16)] = zeros16
            return 0

        lax.fori_loop(0, CH * (hp // 16), fill, 0)

        def zero_acc(i, _):
            pltpu.sync_copy(rows0, acc.at[pl.ds(sid * rpt + i * CH, CH)])
            return 0

        lax.fori_loop(0, rpt // CH, zero_acc, 0)
        plsc.subcore_barrier()
        cbase = (cid * NS + sid) * nch

        def start(idxb, rowsb, semb, c):
            pltpu.sync_copy(eidx_hbm.at[pl.ds(2 * (cbase + c), 2)], idxb)
            pltpu.async_copy(y_hbm.at[idxb.at[0]], rowsb, semb)

        def finish(idxb, rowsb, semb):
            pltpu.make_async_copy(y_hbm.at[idxb.at[0]], rowsb, semb).wait()
            pltpu.sync_copy(rowsb, acc.at[idxb.at[1]], add=True)

        start(idx0, rows0, sem0, 0)

        def body(p, _):
            c = 2 * p
            start(idx1, rows1, sem1, c + 1)
            finish(idx0, rows0, sem0)
            start(idx0, rows0, sem0, c + 2)
            finish(idx1, rows1, sem1)
            return 0

        lax.fori_loop(0, nch // 2 - 1, body, 0)
        start(idx1, rows1, sem1, nch - 1)
        finish(idx0, rows0, sem0)
        finish(idx1, rows1, sem1)

        plsc.subcore_barrier()
        pltpu.sync_copy(
            acc.at[pl.ds(sid * rpt, rpt)],
            out_hbm.at[cid, pl.ds(sid * rpt, rpt)],
        )

    return k


def _tc_stage0(x_pad, w1p, degp):
    n_pad = x_pad.shape[0]
    hp = w1p.shape[1]

    def body(x_ref, w_ref, deg_ref, y_ref, dinv_ref):
        deg = deg_ref[0, :, 0:DEGW] + deg_ref[1, :, 0:DEGW] + 1.0
        dinv = lax.rsqrt(deg)
        dinv_ref[...] = dinv
        xw = jnp.dot(x_ref[...], w_ref[...], preferred_element_type=jnp.float32)
        y_ref[...] = dinv[:, 0:1] * xw

    return pl.pallas_call(
        body,
        out_shape=(
            jax.ShapeDtypeStruct((n_pad, hp), jnp.float32),
            jax.ShapeDtypeStruct((n_pad, DEGW), jnp.float32),
        ),
    )(x_pad, w1p, degp)


def _tc_mid(zp, y, dinv16, bp, wnext):
    n_pad, hp = y.shape

    def body(z_ref, y_ref, dinv_ref, b_ref, w_ref, out_ref):
        dinv = dinv_ref[:, 0:1]
        z = z_ref[0] + z_ref[1] + y_ref[...]
        h = jnp.maximum(dinv * z + b_ref[...], 0.0)
        out_ref[...] = dinv * jnp.dot(h, w_ref[...],
                                      preferred_element_type=jnp.float32)

    return pl.pallas_call(
        body,
        out_shape=jax.ShapeDtypeStruct((n_pad, hp), jnp.float32),
    )(zp, y, dinv16, bp, wnext)


def _tc_stage3(zp, y, dinv16, bp, l1w, l1b, l2w, l2b, n_real):
    n_pad, hp = y.shape
    cp = l2w.shape[1]

    def body(z_ref, y_ref, dinv_ref, b_ref, w1_ref, b1_ref, w2_ref, b2_ref,
             out_ref):
        dinv = dinv_ref[:, 0:1]
        z = z_ref[0] + z_ref[1] + y_ref[...]
        h = jnp.maximum(dinv * z + b_ref[...], 0.0)
        ridx = lax.broadcasted_iota(jnp.int32, h.shape, 0)
        h = jnp.where(ridx < n_real, h, 0.0)
        g = jnp.sum(h, axis=0, keepdims=True)
        g1 = jnp.maximum(
            jnp.dot(g, w1_ref[...], preferred_element_type=jnp.float32)
            + b1_ref[...], 0.0)
        out_ref[...] = (jnp.dot(g1, w2_ref[...],
                                preferred_element_type=jnp.float32)
                        + b2_ref[...])

    return pl.pallas_call(
        body,
        out_shape=jax.ShapeDtypeStruct((1, cp), jnp.float32),
    )(zp, y, dinv16, bp, l1w, l1b, l2w, l2b)


def kernel(x, edge_index, W1, b1, W2, b2, W3, b3, lin1_W, lin1_b, lin2_W,
           lin2_b):
    n, _ = x.shape
    e = edge_index.shape[1]
    h = W1.shape[1]
    h2 = lin1_W.shape[1]
    c = lin2_W.shape[1]

    n_pad = ((n + NS * CH - 1) // (NS * CH)) * (NS * CH)
    hp = ((h + 127) // 128) * 128
    eq = 2 * NC * NS * CH
    e_pad = ((e + eq - 1) // eq) * eq
    h2p = ((h2 + 7) // 8) * 8
    cp = ((c + 7) // 8) * 8

    x_pad = jnp.pad(x, ((0, n_pad - n), (0, 0)))
    pad_cnt = e_pad - e
    pad_idx = (n + jnp.arange(pad_cnt, dtype=jnp.int32) % (n_pad - n)
               ).astype(jnp.int32)
    src = jnp.concatenate([edge_index[0], pad_idx])
    dst = jnp.concatenate([edge_index[1], pad_idx])
    eidx = jnp.stack([src.reshape(-1, CH), dst.reshape(-1, CH)],
                     axis=1).reshape(-1, CH)
    w1p = jnp.pad(W1, ((0, 0), (0, hp - h)))
    w2p = jnp.pad(W2, ((0, hp - h), (0, hp - h)))
    w3p = jnp.pad(W3, ((0, hp - h), (0, hp - h)))
    b1p = jnp.pad(b1, (0, hp - h)).reshape(1, hp)
    b2p = jnp.pad(b2, (0, hp - h)).reshape(1, hp)
    b3p = jnp.pad(b3, (0, hp - h)).reshape(1, hp)
    l1wp = jnp.pad(lin1_W, ((0, hp - h), (0, h2p - h2)))
    l1bp = jnp.pad(lin1_b, (0, h2p - h2)).reshape(1, h2p)
    l2wp = jnp.pad(lin2_W, ((0, h2p - h2), (0, cp - c)))
    l2bp = jnp.pad(lin2_b, (0, cp - c)).reshape(1, cp)

    scatter = _sc_scatter(n_pad, e_pad, hp)

    degp = _sc_degree(n_pad, e_pad, hp)(dst)
    y1, dinv16 = _tc_stage0(x_pad, w1p, degp)
    z1 = scatter(y1, eidx)
    y2 = _tc_mid(z1, y1, dinv16, b1p, w2p)
    z2 = scatter(y2, eidx)
    y3 = _tc_mid(z2, y2, dinv16, b2p, w3p)
    z3 = scatter(y3, eidx)
    outp = _tc_stage3(z3, y3, dinv16, b3p, l1wp, l1bp, l2wp, l2bp, n)
    return outp[:, :c]


# speedup vs baseline: 22.9928x; 1.0328x over previous
import functools

import jax
import jax.numpy as jnp
from jax import lax
from jax.experimental import pallas as pl
from jax.experimental.pallas import tpu as pltpu
from jax.experimental.pallas import tpu_sc as plsc

NC = 2
NS = 16
CH = 64
DEGW = 16


def _sc_degree(n_pad: int, e_pad: int, hp: int):
    ept = e_pad // (NC * NS)
    nch = ept // CH
    rpt = n_pad // NS
    mesh = plsc.VectorSubcoreMesh(core_axis_name="c", subcore_axis_name="s")

    @functools.partial(
        pl.kernel,
        out_type=jax.ShapeDtypeStruct((NC, n_pad, hp), jnp.float32),
        mesh=mesh,
        scratch_types=[
            pltpu.VMEM((e_pad // (NC * NS * CH), CH), jnp.int32),
            pltpu.VMEM((CH, hp), jnp.float32),
            pltpu.VMEM_SHARED((n_pad, hp), jnp.float32),
        ],
    )
    def k(dst_hbm, out_hbm, ibuf, rows, acc, *, nch=nch, rpt=rpt, ept=ept):
        cid = lax.axis_index("c")
        sid = lax.axis_index("s")
        zeros16 = jnp.zeros((16,), jnp.float32)
        ones16 = jnp.full((16,), 1.0, jnp.float32)

        def fill(val):
            def f(i, _):
                rows[i // (hp // 16), pl.ds((i % (hp // 16)) * 16, 16)] = val
                return 0
            lax.fori_loop(0, CH * (hp // 16), f, 0)

        fill(zeros16)

        def zero_acc(i, _):
            pltpu.sync_copy(rows.at[pl.ds(0, 64)],
                            acc.at[pl.ds(sid * rpt + i * 64, 64)])
            return 0

        lax.fori_loop(0, rpt // 64, zero_acc, 0)
        fill(ones16)
        cbase = (cid * NS + sid) * nch

        pltpu.sync_copy(dst_hbm.at[pl.ds(cbase, nch)], ibuf)
        plsc.subcore_barrier()

        def body(ci, _):
            pltpu.sync_copy(rows, acc.at[ibuf.at[ci]], add=True)
            return 0

        lax.fori_loop(0, nch, body, 0)
        plsc.subcore_barrier()
        pltpu.sync_copy(
            acc.at[pl.ds(sid * rpt, rpt)],
            out_hbm.at[cid, pl.ds(sid * rpt, rpt)],
        )

    return k


def _sc_scatter(n_pad: int, e_pad: int, hp: int):
    ept = e_pad // (NC * NS)
    nch = ept // CH
    rpt = n_pad // NS
    mesh = plsc.VectorSubcoreMesh(core_axis_name="c", subcore_axis_name="s")

    @functools.partial(
        pl.kernel,
        out_type=jax.ShapeDtypeStruct((NC, n_pad, hp), jnp.float32),
        mesh=mesh,
        scratch_types=[
            pltpu.VMEM((e_pad // (NC * NS * CH), 2 * CH), jnp.int32),
            pltpu.VMEM((CH, hp), jnp.float32),
            pltpu.VMEM((CH, hp), jnp.float32),
            pltpu.VMEM_SHARED((n_pad, hp), jnp.float32),
            pltpu.SemaphoreType.DMA,
            pltpu.SemaphoreType.DMA,
        ],
    )
    def k(y_hbm, eidx_hbm, out_hbm, ibuf, rows0, rows1, acc,
          sem0, sem1, *, nch=nch, rpt=rpt):
        cid = lax.axis_index("c")
        sid = lax.axis_index("s")
        zeros16 = jnp.zeros((16,), jnp.float32)

        def fill(i, _):
            rows0[i // (hp // 16), pl.ds((i % (hp // 16)) * 16, 16)] = zeros16
            return 0

        lax.fori_loop(0, CH * (hp // 16), fill, 0)

        def zero_acc(i, _):
            pltpu.sync_copy(rows0.at[pl.ds(0, 64)],
                            acc.at[pl.ds(sid * rpt + i * 64, 64)])
            return 0

        lax.fori_loop(0, rpt // 64, zero_acc, 0)
        cbase = (cid * NS + sid) * nch

        pltpu.sync_copy(eidx_hbm.at[pl.ds(cbase, nch)], ibuf)
        plsc.subcore_barrier()

        def start(rowsb, semb, c):
            pltpu.async_copy(y_hbm.at[ibuf.at[c, pl.ds(0, CH)]], rowsb, semb)

        def finish(rowsb, semb, c):
            pltpu.make_async_copy(
                y_hbm.at[ibuf.at[c, pl.ds(0, CH)]], rowsb, semb).wait()
            pltpu.sync_copy(rowsb, acc.at[ibuf.at[c, pl.ds(CH, CH)]],
                            add=True)

        start(rows0, sem0, 0)

        def body(p, _):
            c = 2 * p
            start(rows1, sem1, c + 1)
            finish(rows0, sem0, c)

            @pl.when(c + 2 < nch)
            def _():
                start(rows0, sem0, c + 2)

            finish(rows1, sem1, c + 1)
            return 0

        lax.fori_loop(0, nch // 2, body, 0)

        plsc.subcore_barrier()
        pltpu.sync_copy(
            acc.at[pl.ds(sid * rpt, rpt)],
            out_hbm.at[cid, pl.ds(sid * rpt, rpt)],
        )

    return k


def _tc_stage0(x_pad, w1p, degp):
    n_pad = x_pad.shape[0]
    hp = w1p.shape[1]

    def body(x_ref, w_ref, deg_ref, y_ref, dinv_ref):
        deg = deg_ref[0, :, 0:DEGW] + deg_ref[1, :, 0:DEGW] + 1.0
        dinv = lax.rsqrt(deg)
        dinv_ref[...] = dinv
        xw = jnp.dot(x_ref[...], w_ref[...], preferred_element_type=jnp.float32)
        y_ref[...] = dinv[:, 0:1] * xw

    return pl.pallas_call(
        body,
        out_shape=(
            jax.ShapeDtypeStruct((n_pad, hp), jnp.float32),
            jax.ShapeDtypeStruct((n_pad, DEGW), jnp.float32),
        ),
    )(x_pad, w1p, degp)


def _tc_mid(zp, y, dinv16, bp, wnext):
    n_pad, hp = y.shape

    def body(z_ref, y_ref, dinv_ref, b_ref, w_ref, out_ref):
        dinv = dinv_ref[:, 0:1]
        z = z_ref[0] + z_ref[1] + y_ref[...]
        h = jnp.maximum(dinv * z + b_ref[...], 0.0)
        out_ref[...] = dinv * jnp.dot(h, w_ref[...],
                                      preferred_element_type=jnp.float32)

    return pl.pallas_call(
        body,
        out_shape=jax.ShapeDtypeStruct((n_pad, hp), jnp.float32),
    )(zp, y, dinv16, bp, wnext)


def _tc_stage3(zp, y, dinv16, bp, l1w, l1b, l2w, l2b, n_real):
    n_pad, hp = y.shape
    cp = l2w.shape[1]

    def body(z_ref, y_ref, dinv_ref, b_ref, w1_ref, b1_ref, w2_ref, b2_ref,
             out_ref):
        dinv = dinv_ref[:, 0:1]
        z = z_ref[0] + z_ref[1] + y_ref[...]
        h = jnp.maximum(dinv * z + b_ref[...], 0.0)
        ridx = lax.broadcasted_iota(jnp.int32, h.shape, 0)
        h = jnp.where(ridx < n_real, h, 0.0)
        g = jnp.sum(h, axis=0, keepdims=True)
        g1 = jnp.maximum(
            jnp.dot(g, w1_ref[...], preferred_element_type=jnp.float32)
            + b1_ref[...], 0.0)
        out_ref[...] = (jnp.dot(g1, w2_ref[...],
                                preferred_element_type=jnp.float32)
                        + b2_ref[...])

    return pl.pallas_call(
        body,
        out_shape=jax.ShapeDtypeStruct((1, cp), jnp.float32),
    )(zp, y, dinv16, bp, l1w, l1b, l2w, l2b)


def kernel(x, edge_index, W1, b1, W2, b2, W3, b3, lin1_W, lin1_b, lin2_W,
           lin2_b):
    n, _ = x.shape
    e = edge_index.shape[1]
    h = W1.shape[1]
    h2 = lin1_W.shape[1]
    c = lin2_W.shape[1]

    n_pad = ((n + NS * CH - 1) // (NS * CH)) * (NS * CH)
    hp = ((h + 127) // 128) * 128
    eq = 8 * NC * NS * CH
    e_pad = ((e + eq - 1) // eq) * eq
    h2p = ((h2 + 7) // 8) * 8
    cp = ((c + 7) // 8) * 8

    x_pad = jnp.pad(x, ((0, n_pad - n), (0, 0)))
    pad_cnt = e_pad - e
    pad_idx = (n + jnp.arange(pad_cnt, dtype=jnp.int32) % (n_pad - n)
               ).astype(jnp.int32)
    src = jnp.concatenate([edge_index[0], pad_idx])
    dst = jnp.concatenate([edge_index[1], pad_idx])
    eidx = jnp.concatenate([src.reshape(-1, CH), dst.reshape(-1, CH)],
                           axis=1)
    w1p = jnp.pad(W1, ((0, 0), (0, hp - h)))
    w2p = jnp.pad(W2, ((0, hp - h), (0, hp - h)))
    w3p = jnp.pad(W3, ((0, hp - h), (0, hp - h)))
    b1p = jnp.pad(b1, (0, hp - h)).reshape(1, hp)
    b2p = jnp.pad(b2, (0, hp - h)).reshape(1, hp)
    b3p = jnp.pad(b3, (0, hp - h)).reshape(1, hp)
    l1wp = jnp.pad(lin1_W, ((0, hp - h), (0, h2p - h2)))
    l1bp = jnp.pad(lin1_b, (0, h2p - h2)).reshape(1, h2p)
    l2wp = jnp.pad(lin2_W, ((0, h2p - h2), (0, cp - c)))
    l2bp = jnp.pad(lin2_b, (0, cp - c)).reshape(1, cp)

    scatter = _sc_scatter(n_pad, e_pad, hp)

    degp = _sc_degree(n_pad, e_pad, hp)(dst.reshape(-1, CH))
    y1, dinv16 = _tc_stage0(x_pad, w1p, degp)
    z1 = scatter(y1, eidx)
    y2 = _tc_mid(z1, y1, dinv16, b1p, w2p)
    z2 = scatter(y2, eidx)
    y3 = _tc_mid(z2, y2, dinv16, b2p, w3p)
    z3 = scatter(y3, eidx)
    outp = _tc_stage3(z3, y3, dinv16, b3p, l1wp, l1bp, l2wp, l2bp, n)
    return outp[:, :c]


# speedup vs baseline: 27.4179x; 1.1925x over previous
import functools

import jax
import jax.numpy as jnp
from jax import lax
from jax.experimental import pallas as pl
from jax.experimental.pallas import tpu as pltpu
from jax.experimental.pallas import tpu_sc as plsc

NC = 2
NS = 16
CH = 64
DEGW = 16


def _sc_degree(n_pad: int, e_pad: int, hp: int):
    ept = e_pad // (NC * NS)
    nch = ept // CH
    rpt = n_pad // NS
    mesh = plsc.VectorSubcoreMesh(core_axis_name="c", subcore_axis_name="s")

    @functools.partial(
        pl.kernel,
        out_type=jax.ShapeDtypeStruct((NC, n_pad, hp), jnp.float32),
        mesh=mesh,
        scratch_types=[
            pltpu.VMEM((e_pad // (NC * NS * CH), CH), jnp.int32),
            pltpu.VMEM((CH, hp), jnp.float32),
            pltpu.VMEM_SHARED((n_pad, hp), jnp.float32),
        ],
    )
    def k(dst_hbm, out_hbm, ibuf, rows, acc, *, nch=nch, rpt=rpt, ept=ept):
        cid = lax.axis_index("c")
        sid = lax.axis_index("s")
        zeros16 = jnp.zeros((16,), jnp.float32)
        ones16 = jnp.full((16,), 1.0, jnp.float32)

        def fill(val):
            def f(i, _):
                rows[i // (hp // 16), pl.ds((i % (hp // 16)) * 16, 16)] = val
                return 0
            lax.fori_loop(0, CH * (hp // 16), f, 0)

        fill(zeros16)

        def zero_acc(i, _):
            pltpu.sync_copy(rows.at[pl.ds(0, 64)],
                            acc.at[pl.ds(sid * rpt + i * 64, 64)])
            return 0

        lax.fori_loop(0, rpt // 64, zero_acc, 0)
        fill(ones16)
        cbase = (cid * NS + sid) * nch

        pltpu.sync_copy(dst_hbm.at[pl.ds(cbase, nch)], ibuf)
        plsc.subcore_barrier()

        def body(ci, _):
            pltpu.sync_copy(rows, acc.at[ibuf.at[ci]], add=True)
            return 0

        lax.fori_loop(0, nch, body, 0)
        plsc.subcore_barrier()
        pltpu.sync_copy(
            acc.at[pl.ds(sid * rpt, rpt)],
            out_hbm.at[cid, pl.ds(sid * rpt, rpt)],
        )

    return k


def _sc_scatter(n_pad: int, e_pad: int, hp: int):
    ept = e_pad // (NC * NS)
    nch = ept // CH
    rpt = n_pad // NS
    mesh = plsc.VectorSubcoreMesh(core_axis_name="c", subcore_axis_name="s")

    @functools.partial(
        pl.kernel,
        out_type=jax.ShapeDtypeStruct((NC, n_pad, hp), jnp.float32),
        mesh=mesh,
        scratch_types=[
            pltpu.VMEM((e_pad // (NC * NS * CH), 2 * CH), jnp.int32),
            pltpu.VMEM((CH, hp), jnp.float32),
            pltpu.VMEM((CH, hp), jnp.float32),
            pltpu.VMEM((CH, hp), jnp.float32),
            pltpu.VMEM_SHARED((n_pad, hp), jnp.float32),
            pltpu.SemaphoreType.DMA,
            pltpu.SemaphoreType.DMA,
            pltpu.SemaphoreType.DMA,
        ],
    )
    def k(y_hbm, eidx_hbm, out_hbm, ibuf, rows0, rows1, rows2, acc,
          sem0, sem1, sem2, *, nch=nch, rpt=rpt):
        cid = lax.axis_index("c")
        sid = lax.axis_index("s")
        zeros16 = jnp.zeros((16,), jnp.float32)

        def fill(i, _):
            rows0[i // (hp // 16), pl.ds((i % (hp // 16)) * 16, 16)] = zeros16
            return 0

        lax.fori_loop(0, CH * (hp // 16), fill, 0)

        def zero_acc(i, _):
            pltpu.sync_copy(rows0.at[pl.ds(0, 64)],
                            acc.at[pl.ds(sid * rpt + i * 64, 64)])
            return 0

        lax.fori_loop(0, rpt // 64, zero_acc, 0)
        cbase = (cid * NS + sid) * nch

        pltpu.sync_copy(eidx_hbm.at[pl.ds(cbase, nch)], ibuf)
        plsc.subcore_barrier()

        def start(rowsb, semb, c):
            pltpu.async_copy(y_hbm.at[ibuf.at[c, pl.ds(0, CH)]], rowsb, semb)

        def finish(rowsb, semb, c):
            pltpu.make_async_copy(
                y_hbm.at[ibuf.at[c, pl.ds(0, CH)]], rowsb, semb).wait()
            pltpu.sync_copy(rowsb, acc.at[ibuf.at[c, pl.ds(CH, CH)]],
                            add=True)

        bufs = ((rows0, sem0), (rows1, sem1), (rows2, sem2))
        for r in range(3):
            start(bufs[r][0], bufs[r][1], r)

        def body(p, _):
            c = 3 * p
            for r in range(3):
                rb, sb = bufs[r]

                @pl.when(c + r < nch)
                def _(rb=rb, sb=sb, cc=c + r):
                    finish(rb, sb, cc)

                @pl.when(c + r + 3 < nch)
                def _(rb=rb, sb=sb, cc=c + r + 3):
                    start(rb, sb, cc)
            return 0

        lax.fori_loop(0, (nch + 2) // 3, body, 0)

        plsc.subcore_barrier()
        pltpu.sync_copy(
            acc.at[pl.ds(sid * rpt, rpt)],
            out_hbm.at[cid, pl.ds(sid * rpt, rpt)],
        )

    return k


def _tc_stage0(x_pad, w1p, degp):
    n_pad = x_pad.shape[0]
    hp = w1p.shape[1]

    def body(x_ref, w_ref, deg_ref, y_ref, dinv_ref):
        deg = deg_ref[0, :, 0:DEGW] + deg_ref[1, :, 0:DEGW] + 1.0
        dinv = lax.rsqrt(deg)
        dinv_ref[...] = dinv
        xw = jnp.dot(x_ref[...], w_ref[...], preferred_element_type=jnp.float32)
        y_ref[...] = dinv[:, 0:1] * xw

    return pl.pallas_call(
        body,
        out_shape=(
            jax.ShapeDtypeStruct((n_pad, hp), jnp.float32),
            jax.ShapeDtypeStruct((n_pad, DEGW), jnp.float32),
        ),
    )(x_pad, w1p, degp)


def _tc_mid(zp, y, dinv16, bp, wnext):
    n_pad, hp = y.shape

    def body(z_ref, y_ref, dinv_ref, b_ref, w_ref, out_ref):
        dinv = dinv_ref[:, 0:1]
        z = z_ref[0] + z_ref[1] + y_ref[...]
        h = jnp.maximum(dinv * z + b_ref[...], 0.0)
        out_ref[...] = dinv * jnp.dot(h, w_ref[...],
                                      preferred_element_type=jnp.float32)

    return pl.pallas_call(
        body,
        out_shape=jax.ShapeDtypeStruct((n_pad, hp), jnp.float32),
    )(zp, y, dinv16, bp, wnext)


def _tc_stage3(zp, y, dinv16, bp, l1w, l1b, l2w, l2b, n_real):
    n_pad, hp = y.shape
    cp = l2w.shape[1]

    def body(z_ref, y_ref, dinv_ref, b_ref, w1_ref, b1_ref, w2_ref, b2_ref,
             out_ref):
        dinv = dinv_ref[:, 0:1]
        z = z_ref[0] + z_ref[1] + y_ref[...]
        h = jnp.maximum(dinv * z + b_ref[...], 0.0)
        ridx = lax.broadcasted_iota(jnp.int32, h.shape, 0)
        h = jnp.where(ridx < n_real, h, 0.0)
        g = jnp.sum(h, axis=0, keepdims=True)
        g1 = jnp.maximum(
            jnp.dot(g, w1_ref[...], preferred_element_type=jnp.float32)
            + b1_ref[...], 0.0)
        out_ref[...] = (jnp.dot(g1, w2_ref[...],
                                preferred_element_type=jnp.float32)
                        + b2_ref[...])

    return pl.pallas_call(
        body,
        out_shape=jax.ShapeDtypeStruct((1, cp), jnp.float32),
    )(zp, y, dinv16, bp, l1w, l1b, l2w, l2b)


def kernel(x, edge_index, W1, b1, W2, b2, W3, b3, lin1_W, lin1_b, lin2_W,
           lin2_b):
    n, _ = x.shape
    e = edge_index.shape[1]
    h = W1.shape[1]
    h2 = lin1_W.shape[1]
    c = lin2_W.shape[1]

    n_pad = ((n + NS * CH - 1) // (NS * CH)) * (NS * CH)
    hp = ((h + 127) // 128) * 128
    eq = 8 * NC * NS * CH
    e_pad = ((e + eq - 1) // eq) * eq
    h2p = ((h2 + 7) // 8) * 8
    cp = ((c + 7) // 8) * 8

    x_pad = jnp.pad(x, ((0, n_pad - n), (0, 0)))
    pad_cnt = e_pad - e
    pad_idx = (n + jnp.arange(pad_cnt, dtype=jnp.int32) % (n_pad - n)
               ).astype(jnp.int32)
    src = jnp.concatenate([edge_index[0], pad_idx])
    dst = jnp.concatenate([edge_index[1], pad_idx])
    eidx = jnp.concatenate([src.reshape(-1, CH), dst.reshape(-1, CH)],
                           axis=1)
    w1p = jnp.pad(W1, ((0, 0), (0, hp - h)))
    w2p = jnp.pad(W2, ((0, hp - h), (0, hp - h)))
    w3p = jnp.pad(W3, ((0, hp - h), (0, hp - h)))
    b1p = jnp.pad(b1, (0, hp - h)).reshape(1, hp)
    b2p = jnp.pad(b2, (0, hp - h)).reshape(1, hp)
    b3p = jnp.pad(b3, (0, hp - h)).reshape(1, hp)
    l1wp = jnp.pad(lin1_W, ((0, hp - h), (0, h2p - h2)))
    l1bp = jnp.pad(lin1_b, (0, h2p - h2)).reshape(1, h2p)
    l2wp = jnp.pad(lin2_W, ((0, h2p - h2), (0, cp - c)))
    l2bp = jnp.pad(lin2_b, (0, cp - c)).reshape(1, cp)

    scatter = _sc_scatter(n_pad, e_pad, hp)

    degp = _sc_degree(n_pad, e_pad, hp)(dst.reshape(-1, CH))
    y1, dinv16 = _tc_stage0(x_pad, w1p, degp)
    z1 = scatter(y1, eidx)
    y2 = _tc_mid(z1, y1, dinv16, b1p, w2p)
    z2 = scatter(y2, eidx)
    y3 = _tc_mid(z2, y2, dinv16, b2p, w3p)
    z3 = scatter(y3, eidx)
    outp = _tc_stage3(z3, y3, dinv16, b3p, l1wp, l1bp, l2wp, l2bp, n)
    return outp[:, :c]


# speedup vs baseline: 27.4200x; 1.0001x over previous
import functools

import jax
import jax.numpy as jnp
from jax import lax
from jax.experimental import pallas as pl
from jax.experimental.pallas import tpu as pltpu
from jax.experimental.pallas import tpu_sc as plsc

NC = 2
NS = 16
CH = 64
DEGW = 16


def _sc_degree(n_pad: int, e_pad: int, hp: int):
    ept = e_pad // (NC * NS)
    nch = ept // CH
    rpt = n_pad // NS
    mesh = plsc.VectorSubcoreMesh(core_axis_name="c", subcore_axis_name="s")

    @functools.partial(
        pl.kernel,
        out_type=jax.ShapeDtypeStruct((NC, n_pad, hp), jnp.float32),
        mesh=mesh,
        scratch_types=[
            pltpu.VMEM((e_pad // (NC * NS * CH), CH), jnp.int32),
            pltpu.VMEM((CH, hp), jnp.float32),
            pltpu.VMEM_SHARED((n_pad, hp), jnp.float32),
        ],
    )
    def k(dst_hbm, out_hbm, ibuf, rows, acc, *, nch=nch, rpt=rpt, ept=ept):
        cid = lax.axis_index("c")
        sid = lax.axis_index("s")
        zeros16 = jnp.zeros((16,), jnp.float32)
        ones16 = jnp.full((16,), 1.0, jnp.float32)

        def fill(val):
            def f(i, _):
                rows[i // (hp // 16), pl.ds((i % (hp // 16)) * 16, 16)] = val
                return 0
            lax.fori_loop(0, CH * (hp // 16), f, 0)

        fill(zeros16)

        def zero_acc(i, _):
            pltpu.sync_copy(rows.at[pl.ds(0, 64)],
                            acc.at[pl.ds(sid * rpt + i * 64, 64)])
            return 0

        lax.fori_loop(0, rpt // 64, zero_acc, 0)
        fill(ones16)
        cbase = (cid * NS + sid) * nch

        pltpu.sync_copy(dst_hbm.at[pl.ds(cbase, nch)], ibuf)
        plsc.subcore_barrier()

        def body(ci, _):
            pltpu.sync_copy(rows, acc.at[ibuf.at[ci]], add=True)
            return 0

        lax.fori_loop(0, nch, body, 0)
        plsc.subcore_barrier()
        pltpu.sync_copy(
            acc.at[pl.ds(sid * rpt, rpt)],
            out_hbm.at[cid, pl.ds(sid * rpt, rpt)],
        )

    return k


def _sc_scatter(n_pad: int, e_pad: int, hp: int):
    ept = e_pad // (NC * NS)
    nch = ept // CH
    rpt = n_pad // NS
    mesh = plsc.VectorSubcoreMesh(core_axis_name="c", subcore_axis_name="s")

    @functools.partial(
        pl.kernel,
        out_type=jax.ShapeDtypeStruct((NC, n_pad, hp), jnp.float32),
        mesh=mesh,
        scratch_types=[
            pltpu.VMEM((e_pad // (NC * NS * CH), 2 * CH), jnp.int32),
            pltpu.VMEM((CH, hp), jnp.float32),
            pltpu.VMEM((CH, hp), jnp.float32),
            pltpu.VMEM((CH, hp), jnp.float32),
            pltpu.VMEM_SHARED((n_pad, hp), jnp.float32),
            pltpu.SemaphoreType.DMA,
            pltpu.SemaphoreType.DMA,
            pltpu.SemaphoreType.DMA,
        ],
    )
    def k(y_hbm, eidx_hbm, out_hbm, ibuf, rows0, rows1, rows2, acc,
          sem0, sem1, sem2, *, nch=nch, rpt=rpt):
        cid = lax.axis_index("c")
        sid = lax.axis_index("s")
        zeros16 = jnp.zeros((16,), jnp.float32)

        def fill(i, _):
            rows0[i // (hp // 16), pl.ds((i % (hp // 16)) * 16, 16)] = zeros16
            return 0

        lax.fori_loop(0, CH * (hp // 16), fill, 0)

        def zero_acc(i, _):
            pltpu.sync_copy(rows0.at[pl.ds(0, 64)],
                            acc.at[pl.ds(sid * rpt + i * 64, 64)])
            return 0

        lax.fori_loop(0, rpt // 64, zero_acc, 0)
        cbase = (cid * NS + sid) * nch

        pltpu.sync_copy(eidx_hbm.at[pl.ds(cbase, nch)], ibuf)
        plsc.subcore_barrier()

        def start(rowsb, semb, c):
            pltpu.async_copy(y_hbm.at[ibuf.at[c, pl.ds(0, CH)]], rowsb, semb)

        def finish(rowsb, semb, c):
            pltpu.make_async_copy(
                y_hbm.at[ibuf.at[c, pl.ds(0, CH)]], rowsb, semb).wait()
            pltpu.sync_copy(rowsb, acc.at[ibuf.at[c, pl.ds(CH, CH)]],
                            add=True)

        bufs = ((rows0, sem0), (rows1, sem1), (rows2, sem2))
        for r in range(3):
            start(bufs[r][0], bufs[r][1], r)

        def body(p, _):
            c = 3 * p
            for r in range(3):
                rb, sb = bufs[r]

                @pl.when(c + r < nch)
                def _(rb=rb, sb=sb, cc=c + r):
                    finish(rb, sb, cc)

                @pl.when(c + r + 3 < nch)
                def _(rb=rb, sb=sb, cc=c + r + 3):
                    start(rb, sb, cc)
            return 0

        lax.fori_loop(0, (nch + 2) // 3, body, 0)

        plsc.subcore_barrier()
        pltpu.sync_copy(
            acc.at[pl.ds(sid * rpt, rpt)],
            out_hbm.at[cid, pl.ds(sid * rpt, rpt)],
        )

    return k


def _tc_xw(x_pad, w1p):
    n_pad = x_pad.shape[0]
    hp = w1p.shape[1]

    def body(x_ref, w_ref, o_ref):
        o_ref[...] = jnp.dot(x_ref[...], w_ref[...],
                             preferred_element_type=jnp.float32)

    return pl.pallas_call(
        body,
        out_shape=jax.ShapeDtypeStruct((n_pad, hp), jnp.float32),
    )(x_pad, w1p)


def _tc_stage0(xw, degp):
    n_pad, hp = xw.shape

    def body(xw_ref, deg_ref, y_ref, dinv_ref):
        deg = deg_ref[0, :, 0:DEGW] + deg_ref[1, :, 0:DEGW] + 1.0
        dinv = lax.rsqrt(deg)
        dinv_ref[...] = dinv
        y_ref[...] = dinv[:, 0:1] * xw_ref[...]

    return pl.pallas_call(
        body,
        out_shape=(
            jax.ShapeDtypeStruct((n_pad, hp), jnp.float32),
            jax.ShapeDtypeStruct((n_pad, DEGW), jnp.float32),
        ),
    )(xw, degp)


def _tc_mid(zp, y, dinv16, bp, wnext):
    n_pad, hp = y.shape

    def body(z_ref, y_ref, dinv_ref, b_ref, w_ref, out_ref):
        dinv = dinv_ref[:, 0:1]
        z = z_ref[0] + z_ref[1] + y_ref[...]
        h = jnp.maximum(dinv * z + b_ref[...], 0.0)
        out_ref[...] = dinv * jnp.dot(h, w_ref[...],
                                      preferred_element_type=jnp.float32)

    return pl.pallas_call(
        body,
        out_shape=jax.ShapeDtypeStruct((n_pad, hp), jnp.float32),
    )(zp, y, dinv16, bp, wnext)


def _tc_stage3(zp, y, dinv16, bp, l1w, l1b, l2w, l2b, n_real):
    n_pad, hp = y.shape
    cp = l2w.shape[1]

    def body(z_ref, y_ref, dinv_ref, b_ref, w1_ref, b1_ref, w2_ref, b2_ref,
             out_ref):
        dinv = dinv_ref[:, 0:1]
        z = z_ref[0] + z_ref[1] + y_ref[...]
        h = jnp.maximum(dinv * z + b_ref[...], 0.0)
        ridx = lax.broadcasted_iota(jnp.int32, h.shape, 0)
        h = jnp.where(ridx < n_real, h, 0.0)
        g = jnp.sum(h, axis=0, keepdims=True)
        g1 = jnp.maximum(
            jnp.dot(g, w1_ref[...], preferred_element_type=jnp.float32)
            + b1_ref[...], 0.0)
        out_ref[...] = (jnp.dot(g1, w2_ref[...],
                                preferred_element_type=jnp.float32)
                        + b2_ref[...])

    return pl.pallas_call(
        body,
        out_shape=jax.ShapeDtypeStruct((1, cp), jnp.float32),
    )(zp, y, dinv16, bp, l1w, l1b, l2w, l2b)


def kernel(x, edge_index, W1, b1, W2, b2, W3, b3, lin1_W, lin1_b, lin2_W,
           lin2_b):
    n, _ = x.shape
    e = edge_index.shape[1]
    h = W1.shape[1]
    h2 = lin1_W.shape[1]
    c = lin2_W.shape[1]

    n_pad = ((n + NS * CH - 1) // (NS * CH)) * (NS * CH)
    hp = ((h + 127) // 128) * 128
    eq = 8 * NC * NS * CH
    e_pad = ((e + eq - 1) // eq) * eq
    h2p = ((h2 + 7) // 8) * 8
    cp = ((c + 7) // 8) * 8

    x_pad = jnp.pad(x, ((0, n_pad - n), (0, 0)))
    pad_cnt = e_pad - e
    pad_idx = (n + jnp.arange(pad_cnt, dtype=jnp.int32) % (n_pad - n)
               ).astype(jnp.int32)
    src = jnp.concatenate([edge_index[0], pad_idx])
    dst = jnp.concatenate([edge_index[1], pad_idx])
    eidx = jnp.concatenate([src.reshape(-1, CH), dst.reshape(-1, CH)],
                           axis=1)
    w1p = jnp.pad(W1, ((0, 0), (0, hp - h)))
    w2p = jnp.pad(W2, ((0, hp - h), (0, hp - h)))
    w3p = jnp.pad(W3, ((0, hp - h), (0, hp - h)))
    b1p = jnp.pad(b1, (0, hp - h)).reshape(1, hp)
    b2p = jnp.pad(b2, (0, hp - h)).reshape(1, hp)
    b3p = jnp.pad(b3, (0, hp - h)).reshape(1, hp)
    l1wp = jnp.pad(lin1_W, ((0, hp - h), (0, h2p - h2)))
    l1bp = jnp.pad(lin1_b, (0, h2p - h2)).reshape(1, h2p)
    l2wp = jnp.pad(lin2_W, ((0, h2p - h2), (0, cp - c)))
    l2bp = jnp.pad(lin2_b, (0, cp - c)).reshape(1, cp)

    scatter = _sc_scatter(n_pad, e_pad, hp)

    xw1 = _tc_xw(x_pad, w1p)
    degp = _sc_degree(n_pad, e_pad, hp)(dst.reshape(-1, CH))
    y1, dinv16 = _tc_stage0(xw1, degp)
    z1 = scatter(y1, eidx)
    y2 = _tc_mid(z1, y1, dinv16, b1p, w2p)
    z2 = scatter(y2, eidx)
    y3 = _tc_mid(z2, y2, dinv16, b2p, w3p)
    z3 = scatter(y3, eidx)
    outp = _tc_stage3(z3, y3, dinv16, b3p, l1wp, l1bp, l2wp, l2bp, n)
    return outp[:, :c]


# speedup vs baseline: 28.3091x; 1.0324x over previous
import functools

import jax
import jax.numpy as jnp
from jax import lax
from jax.experimental import pallas as pl
from jax.experimental.pallas import tpu as pltpu
from jax.experimental.pallas import tpu_sc as plsc

NC = 2
NS = 16
CH = 64
DEGW = 16


def _sc_degree(n_pad: int, e_pad: int, hp: int):
    ept = e_pad // (NC * NS)
    nch = ept // CH
    rpt = n_pad // NS
    mesh = plsc.VectorSubcoreMesh(core_axis_name="c", subcore_axis_name="s")

    @functools.partial(
        pl.kernel,
        out_type=jax.ShapeDtypeStruct((NC, n_pad, hp), jnp.float32),
        mesh=mesh,
        scratch_types=[
            pltpu.VMEM((e_pad // (NC * NS * CH), CH), jnp.int32),
            pltpu.VMEM((CH, hp), jnp.float32),
            pltpu.VMEM_SHARED((n_pad, hp), jnp.float32),
        ],
    )
    def k(dst_hbm, out_hbm, ibuf, rows, acc, *, nch=nch, rpt=rpt, ept=ept):
        cid = lax.axis_index("c")
        sid = lax.axis_index("s")
        zeros16 = jnp.zeros((16,), jnp.float32)
        ones16 = jnp.full((16,), 1.0, jnp.float32)

        def fill(val):
            def f(i, _):
                rows[i // (hp // 16), pl.ds((i % (hp // 16)) * 16, 16)] = val
                return 0
            lax.fori_loop(0, CH * (hp // 16), f, 0)

        fill(zeros16)

        def zero_acc(i, _):
            pltpu.sync_copy(rows.at[pl.ds(0, 64)],
                            acc.at[pl.ds(sid * rpt + i * 64, 64)])
            return 0

        lax.fori_loop(0, rpt // 64, zero_acc, 0)
        fill(ones16)
        cbase = (cid * NS + sid) * nch

        pltpu.sync_copy(dst_hbm.at[pl.ds(cbase, nch)], ibuf)
        plsc.subcore_barrier()

        def body(ci, _):
            pltpu.sync_copy(rows, acc.at[ibuf.at[ci]], add=True)
            return 0

        lax.fori_loop(0, nch, body, 0)
        plsc.subcore_barrier()
        pltpu.sync_copy(
            acc.at[pl.ds(sid * rpt, rpt)],
            out_hbm.at[cid, pl.ds(sid * rpt, rpt)],
        )

    return k


def _sc_scatter(n_pad: int, e_pad: int, hp: int):
    ept = e_pad // (NC * NS)
    nch = ept // CH
    rpt = n_pad // NS
    mesh = plsc.VectorSubcoreMesh(core_axis_name="c", subcore_axis_name="s")

    @functools.partial(
        pl.kernel,
        out_type=jax.ShapeDtypeStruct((NC, n_pad, hp), jnp.float32),
        mesh=mesh,
        scratch_types=[
            pltpu.VMEM((e_pad // (2 * NC * NS * CH), 2 * CH), jnp.int32),
            pltpu.VMEM((CH, hp), jnp.float32),
            pltpu.VMEM((CH, hp), jnp.float32),
            pltpu.VMEM((CH, hp), jnp.float32),
            pltpu.VMEM((CH, hp), jnp.float32),
            pltpu.VMEM_SHARED((n_pad, hp), jnp.float32),
            pltpu.SemaphoreType.DMA,
            pltpu.SemaphoreType.DMA,
            pltpu.SemaphoreType.DMA,
            pltpu.SemaphoreType.DMA,
        ],
    )
    def k(y_hbm, eidx_hbm, out_hbm, ibuf, rows0, rows1, rows2, rows3, acc,
          sem0, sem1, sem2, sem3, *, nch=nch, rpt=rpt):
        cid = lax.axis_index("c")
        sid = lax.axis_index("s")
        zeros16 = jnp.zeros((16,), jnp.float32)

        def fill(i, _):
            rows0[i // (hp // 16), pl.ds((i % (hp // 16)) * 16, 16)] = zeros16
            return 0

        lax.fori_loop(0, CH * (hp // 16), fill, 0)

        def zero_acc(i, _):
            pltpu.sync_copy(rows0.at[pl.ds(0, 64)],
                            acc.at[pl.ds(sid * rpt + i * 64, 64)])
            return 0

        lax.fori_loop(0, rpt // 64, zero_acc, 0)
        cbase = (cid * NS + sid) * nch
        nh = nch // 2
        plsc.subcore_barrier()

        def start(rowsb, semb, lc):
            pltpu.async_copy(y_hbm.at[ibuf.at[lc, pl.ds(0, CH)]], rowsb, semb)

        def finish(rowsb, semb, lc):
            pltpu.make_async_copy(
                y_hbm.at[ibuf.at[lc, pl.ds(0, CH)]], rowsb, semb).wait()
            pltpu.sync_copy(rowsb, acc.at[ibuf.at[lc, pl.ds(CH, CH)]],
                            add=True)

        bufs = ((rows0, sem0), (rows1, sem1), (rows2, sem2), (rows3, sem3))
        for h in range(2):
            pltpu.sync_copy(eidx_hbm.at[pl.ds(cbase + h * nh, nh)], ibuf)
            for r in range(4):
                start(bufs[r][0], bufs[r][1], r)

            def body(p, _):
                c = 4 * p
                for r in range(4):
                    rb, sb = bufs[r]
                    finish(rb, sb, c + r)

                    @pl.when(c + r + 4 < nh)
                    def _(rb=rb, sb=sb, lc=c + r + 4):
                        start(rb, sb, lc)
                return 0

            lax.fori_loop(0, nh // 4, body, 0)

        plsc.subcore_barrier()
        pltpu.sync_copy(
            acc.at[pl.ds(sid * rpt, rpt)],
            out_hbm.at[cid, pl.ds(sid * rpt, rpt)],
        )

    return k


def _tc_xw(x_pad, w1p):
    n_pad = x_pad.shape[0]
    hp = w1p.shape[1]

    def body(x_ref, w_ref, o_ref):
        o_ref[...] = jnp.dot(x_ref[...], w_ref[...],
                             preferred_element_type=jnp.float32)

    return pl.pallas_call(
        body,
        out_shape=jax.ShapeDtypeStruct((n_pad, hp), jnp.float32),
    )(x_pad, w1p)


def _tc_stage0(xw, degp):
    n_pad, hp = xw.shape

    def body(xw_ref, deg_ref, y_ref, dinv_ref):
        deg = deg_ref[0, :, 0:DEGW] + deg_ref[1, :, 0:DEGW] + 1.0
        dinv = lax.rsqrt(deg)
        dinv_ref[...] = dinv
        y_ref[...] = dinv[:, 0:1] * xw_ref[...]

    return pl.pallas_call(
        body,
        out_shape=(
            jax.ShapeDtypeStruct((n_pad, hp), jnp.float32),
            jax.ShapeDtypeStruct((n_pad, DEGW), jnp.float32),
        ),
    )(xw, degp)


def _tc_mid(zp, y, dinv16, bp, wnext):
    n_pad, hp = y.shape

    def body(z_ref, y_ref, dinv_ref, b_ref, w_ref, out_ref):
        dinv = dinv_ref[:, 0:1]
        z = z_ref[0] + z_ref[1] + y_ref[...]
        h = jnp.maximum(dinv * z + b_ref[...], 0.0)
        out_ref[...] = dinv * jnp.dot(h, w_ref[...],
                                      preferred_element_type=jnp.float32)

    return pl.pallas_call(
        body,
        out_shape=jax.ShapeDtypeStruct((n_pad, hp), jnp.float32),
    )(zp, y, dinv16, bp, wnext)


def _tc_stage3(zp, y, dinv16, bp, l1w, l1b, l2w, l2b, n_real):
    n_pad, hp = y.shape
    cp = l2w.shape[1]

    def body(z_ref, y_ref, dinv_ref, b_ref, w1_ref, b1_ref, w2_ref, b2_ref,
             out_ref):
        dinv = dinv_ref[:, 0:1]
        z = z_ref[0] + z_ref[1] + y_ref[...]
        h = jnp.maximum(dinv * z + b_ref[...], 0.0)
        ridx = lax.broadcasted_iota(jnp.int32, h.shape, 0)
        h = jnp.where(ridx < n_real, h, 0.0)
        g = jnp.sum(h, axis=0, keepdims=True)
        g1 = jnp.maximum(
            jnp.dot(g, w1_ref[...], preferred_element_type=jnp.float32)
            + b1_ref[...], 0.0)
        out_ref[...] = (jnp.dot(g1, w2_ref[...],
                                preferred_element_type=jnp.float32)
                        + b2_ref[...])

    return pl.pallas_call(
        body,
        out_shape=jax.ShapeDtypeStruct((1, cp), jnp.float32),
    )(zp, y, dinv16, bp, l1w, l1b, l2w, l2b)


def kernel(x, edge_index, W1, b1, W2, b2, W3, b3, lin1_W, lin1_b, lin2_W,
           lin2_b):
    n, _ = x.shape
    e = edge_index.shape[1]
    h = W1.shape[1]
    h2 = lin1_W.shape[1]
    c = lin2_W.shape[1]

    n_pad = ((n + NS * CH - 1) // (NS * CH)) * (NS * CH)
    hp = ((h + 127) // 128) * 128
    eq = 8 * NC * NS * CH
    e_pad = ((e + eq - 1) // eq) * eq
    h2p = ((h2 + 7) // 8) * 8
    cp = ((c + 7) // 8) * 8

    x_pad = jnp.pad(x, ((0, n_pad - n), (0, 0)))
    pad_cnt = e_pad - e
    pad_idx = (n + jnp.arange(pad_cnt, dtype=jnp.int32) % (n_pad - n)
               ).astype(jnp.int32)
    src = jnp.concatenate([edge_index[0], pad_idx])
    dst = jnp.concatenate([edge_index[1], pad_idx])
    eidx = jnp.concatenate([src.reshape(-1, CH), dst.reshape(-1, CH)],
                           axis=1)
    w1p = jnp.pad(W1, ((0, 0), (0, hp - h)))
    w2p = jnp.pad(W2, ((0, hp - h), (0, hp - h)))
    w3p = jnp.pad(W3, ((0, hp - h), (0, hp - h)))
    b1p = jnp.pad(b1, (0, hp - h)).reshape(1, hp)
    b2p = jnp.pad(b2, (0, hp - h)).reshape(1, hp)
    b3p = jnp.pad(b3, (0, hp - h)).reshape(1, hp)
    l1wp = jnp.pad(lin1_W, ((0, hp - h), (0, h2p - h2)))
    l1bp = jnp.pad(lin1_b, (0, h2p - h2)).reshape(1, h2p)
    l2wp = jnp.pad(lin2_W, ((0, h2p - h2), (0, cp - c)))
    l2bp = jnp.pad(lin2_b, (0, cp - c)).reshape(1, cp)

    scatter = _sc_scatter(n_pad, e_pad, hp)

    xw1 = _tc_xw(x_pad, w1p)
    degp = _sc_degree(n_pad, e_pad, hp)(dst.reshape(-1, CH))
    y1, dinv16 = _tc_stage0(xw1, degp)
    z1 = scatter(y1, eidx)
    y2 = _tc_mid(z1, y1, dinv16, b1p, w2p)
    z2 = scatter(y2, eidx)
    y3 = _tc_mid(z2, y2, dinv16, b2p, w3p)
    z3 = scatter(y3, eidx)
    outp = _tc_stage3(z3, y3, dinv16, b3p, l1wp, l1bp, l2wp, l2bp, n)
    return outp[:, :c]
